# baseline (speedup 1.0000x reference)
import functools

import jax
import jax.numpy as jnp
from jax import lax
from jax.experimental import pallas as pl
from jax.experimental.pallas import tpu as pltpu
from jax.experimental.pallas import tpu_sc as plsc

N = 10000
NP = 10240
E = 320000
B = 512
VOCAB = 100000
ED = 128
GH = 256
OUT = 128

EROWS = E // 128
EROWSP = 2560
CH = EROWSP // 16
EPAD = EROWSP * 128 - E
DEAD = N + 16
NB = 512
NBLK = NP // NB
TPC = 16
DPT = NP // TPC
BROWS = NP // 128
CPAD = B + 128


def _sc_stats_body(dstp, batchp, entp, ent_emb, zeros1,
                   deg_out, cnt_out, ee_out,
                   degacc, cntacc, idxb, onesv, rows, entb):
    c = lax.axis_index("c")
    s = lax.axis_index("s")
    for i in range(8):
        onesv[pl.ds(i * 16, 16)] = jnp.ones((16,), jnp.float32)

    @pl.when(s == 0)
    def _():
        pltpu.sync_copy(zeros1, degacc)

    @pl.when(s == 1)
    def _():
        pltpu.sync_copy(zeros1.at[pl.ds(0, CPAD)], cntacc)

    plsc.subcore_barrier()

    def deg_step(k, _):
        r = s + TPC * k
        pltpu.sync_copy(dstp.at[c, r], idxb.at[0])
        pltpu.sync_copy(onesv, degacc.at[idxb.at[0]], add=True)
        return 0

    lax.fori_loop(0, EROWSP // TPC, deg_step, 0)

    def cnt_step(k, _):
        r = s + TPC * k
        pltpu.sync_copy(batchp.at[c, r], idxb.at[0])
        pltpu.sync_copy(onesv, cntacc.at[idxb.at[0]], add=True)
        return 0

    nitb = (BROWS - s + TPC - 1) // TPC
    lax.fori_loop(0, nitb, cnt_step, 0)

    @pl.when(s < 4)
    def _():
        pltpu.sync_copy(entp.at[c, s], entb)
        pltpu.sync_copy(ent_emb.at[entb], rows)
        pltpu.sync_copy(rows, ee_out.at[c, pl.ds(s * 128, 128)])

    plsc.subcore_barrier()
    pltpu.sync_copy(degacc.at[pl.ds(s * DPT, DPT)],
                    deg_out.at[c, pl.ds(s * DPT, DPT)])

    @pl.when(s == 0)
    def _():
        pltpu.sync_copy(cntacc, cnt_out.at[c])


@functools.lru_cache(maxsize=None)
def _sc_stats():
    return pl.kernel(
        _sc_stats_body,
        out_type=(
            jax.ShapeDtypeStruct((2, NP), jnp.float32),
            jax.ShapeDtypeStruct((2, CPAD), jnp.float32),
            jax.ShapeDtypeStruct((2, B, ED), jnp.float32),
        ),
        mesh=plsc.VectorSubcoreMesh(core_axis_name="c", subcore_axis_name="s"),
        scratch_types=[
            pltpu.VMEM_SHARED((NP,), jnp.float32),
            pltpu.VMEM_SHARED((CPAD,), jnp.float32),
            pltpu.VMEM((1, 128), jnp.int32),
            pltpu.VMEM((128,), jnp.float32),
            pltpu.VMEM((128, 128), jnp.float32),
            pltpu.VMEM((128,), jnp.int32),
        ],
    )


IBLK = 32


def _sc_edge_body(srcp, dstp, hs_all, zeros2,
                  acc_out,
                  accS, sidxb, didxb, rows0, rows1,
                  g0, g1, s0, s1):
    c = lax.axis_index("c")
    s = lax.axis_index("s")
    rows = (rows0, rows1)
    gsem = (g0, g1)
    ssem = (s0, s1)
    start = CH * s

    def gather_start(k, b):
        pltpu.async_copy(hs_all.at[sidxb.at[k]], rows[b], gsem[b])

    def gather_wait(k, b):
        pltpu.make_async_copy(hs_all.at[sidxb.at[k]], rows[b],
                              gsem[b]).wait()

    def scat_start(k, b):
        pltpu.async_copy(rows[b], accS.at[didxb.at[k]], ssem[b], add=True)

    def scat_wait(k, b):
        pltpu.make_async_copy(rows[b], accS.at[didxb.at[k]], ssem[b]).wait()

    def graph_pass(g, _):
        for z in range(DPT // 128):
            pltpu.sync_copy(zeros2,
                            accS.at[pl.ds(s * DPT + z * 128, 128)])
        plsc.subcore_barrier()

        def block(blk, _):
            base = start + blk * IBLK
            pltpu.sync_copy(srcp.at[g, c, pl.ds(base, IBLK)], sidxb)
            pltpu.sync_copy(dstp.at[g, pl.ds(base, IBLK)], didxb)

            def inner(k, _):
                pltpu.sync_copy(hs_all.at[sidxb.at[k]], rows0)
                pltpu.sync_copy(rows0, accS.at[didxb.at[k]], add=True)
                return 0

            lax.fori_loop(0, IBLK, inner, 0)
            return 0

        lax.fori_loop(0, CH // IBLK, block, 0)

        plsc.subcore_barrier()
        for z in range(DPT // 128):
            pltpu.sync_copy(accS.at[pl.ds(s * DPT + z * 128, 128)],
                            acc_out.at[g, c, pl.ds(s * DPT + z * 128, 128)])
        plsc.subcore_barrier()
        return 0

    lax.fori_loop(0, 2, graph_pass, 0)


@functools.lru_cache(maxsize=None)
def _sc_edge():
    return pl.kernel(
        _sc_edge_body,
        out_type=jax.ShapeDtypeStruct((2, 2, NP, 128), jnp.float32),
        mesh=plsc.VectorSubcoreMesh(core_axis_name="c", subcore_axis_name="s"),
        scratch_types=[
            pltpu.VMEM_SHARED((NP, 128), jnp.float32),
            pltpu.VMEM((IBLK, 128), jnp.int32),
            pltpu.VMEM((IBLK, 128), jnp.int32),
            pltpu.VMEM((128, 128), jnp.float32),
            pltpu.VMEM((128, 128), jnp.float32),
            pltpu.SemaphoreType.DMA,
            pltpu.SemaphoreType.DMA,
            pltpu.SemaphoreType.DMA,
            pltpu.SemaphoreType.DMA,
        ],
    )


def _tc1_body(xs_ref, deg_ref, aemb_ref, gW1_ref, hs_ref):
    x = xs_ref[0]
    iota = lax.broadcasted_iota(jnp.int32, (NB, 16), 1).astype(jnp.float32)
    onehot = (x == iota).astype(jnp.float32)
    weff = jnp.dot(aemb_ref[...], gW1_ref[...],
                   preferred_element_type=jnp.float32)
    h = jnp.dot(onehot, weff, preferred_element_type=jnp.float32)
    dinv = lax.rsqrt(deg_ref[0] + 1.0)
    hs = h * dinv
    hs_ref[0, 0] = hs[:, :128]
    hs_ref[0, 1] = hs[:, 128:]


def _tc1(xs, degc, aemb, gW1):
    return pl.pallas_call(
        _tc1_body,
        grid=(2, NBLK),
        in_specs=[
            pl.BlockSpec((1, NB, 1), lambda g, r: (g, r, 0)),
            pl.BlockSpec((1, NB, 1), lambda g, r: (g, r, 0)),
            pl.BlockSpec((16, ED), lambda g, r: (0, 0)),
            pl.BlockSpec((ED, GH), lambda g, r: (0, 0)),
        ],
        out_specs=pl.BlockSpec((1, 2, NB, 128), lambda g, r: (g, 0, r, 0)),
        out_shape=jax.ShapeDtypeStruct((2, 2, NP, 128), jnp.float32),
    )(xs, degc, aemb, gW1)


def _tc3_body(acc_ref, hs_ref, deg_ref, b1_ref, gW2_ref, out_ref):
    dinv = lax.rsqrt(deg_ref[0] + 1.0)
    a = jnp.concatenate([acc_ref[0, 0], acc_ref[0, 1]], axis=1)
    hsv = jnp.concatenate([hs_ref[0, 0], hs_ref[0, 1]], axis=1)
    out1 = jnp.maximum((a + hsv) * dinv + b1_ref[...], 0.0)
    h2 = jnp.dot(out1, gW2_ref[...], preferred_element_type=jnp.float32)
    h2s = h2 * dinv
    out_ref[0, 0] = h2s[:, :128]
    out_ref[0, 1] = h2s[:, 128:]


def _tc3(acc1, hs1, degc, b1, gW2):
    return pl.pallas_call(
        _tc3_body,
        grid=(2, NBLK),
        in_specs=[
            pl.BlockSpec((1, 2, NB, 128), lambda g, r: (g, 0, r, 0)),
            pl.BlockSpec((1, 2, NB, 128), lambda g, r: (g, 0, r, 0)),
            pl.BlockSpec((1, NB, 1), lambda g, r: (g, r, 0)),
            pl.BlockSpec((1, GH), lambda g, r: (0, 0)),
            pl.BlockSpec((GH, GH), lambda g, r: (0, 0)),
        ],
        out_specs=pl.BlockSpec((1, 2, NB, 128), lambda g, r: (g, 0, r, 0)),
        out_shape=jax.ShapeDtypeStruct((2, 2, NP, 128), jnp.float32),
    )(acc1, hs1, degc, b1, gW2)


def _tc5_body(acc_ref, hs_ref, deg_ref, b2_ref, batch_ref, sums_ref):
    r = pl.program_id(1)
    dinv = lax.rsqrt(deg_ref[0] + 1.0)
    a = jnp.concatenate([acc_ref[0, 0], acc_ref[0, 1]], axis=1)
    hsv = jnp.concatenate([hs_ref[0, 0], hs_ref[0, 1]], axis=1)
    out2 = jnp.maximum((a + hsv) * dinv + b2_ref[...], 0.0)
    iota = lax.broadcasted_iota(jnp.int32, (NB, B), 1).astype(jnp.float32)
    onehot = (batch_ref[0] == iota).astype(jnp.float32)
    part = lax.dot_general(onehot, out2, (((0,), (0,)), ((), ())),
                           preferred_element_type=jnp.float32)

    @pl.when(r == 0)
    def _():
        sums_ref[0] = jnp.zeros((B, GH), jnp.float32)

    sums_ref[0] += part


def _tc5(acc2, hs2, degc, b2, batchf):
    return pl.pallas_call(
        _tc5_body,
        grid=(2, NBLK),
        in_specs=[
            pl.BlockSpec((1, 2, NB, 128), lambda g, r: (g, 0, r, 0)),
            pl.BlockSpec((1, 2, NB, 128), lambda g, r: (g, 0, r, 0)),
            pl.BlockSpec((1, NB, 1), lambda g, r: (g, r, 0)),
            pl.BlockSpec((1, GH), lambda g, r: (0, 0)),
            pl.BlockSpec((1, NB, 1), lambda g, r: (g, r, 0)),
        ],
        out_specs=pl.BlockSpec((1, B, GH), lambda g, r: (g, 0, 0)),
        out_shape=jax.ShapeDtypeStruct((2, B, GH), jnp.float32),
    )(acc2, hs2, degc, b2, batchf)


def _tc7_body(sums_ref, cnt_ref, ee_ref, fcW_ref, fcb_ref,
              eW1_ref, eb1_ref, eW2_ref, eb2_ref,
              dW1_ref, db1_ref, dW2_ref, db2_ref, dW3_ref, db3_ref, o_ref):
    dot = functools.partial(jnp.dot, preferred_element_type=jnp.float32)
    gs = []
    es = []
    for g in range(2):
        pooled = sums_ref[g] / jnp.maximum(cnt_ref[g], 1.0)
        gs.append(dot(pooled, fcW_ref[...]) + fcb_ref[...])
        e = jnp.maximum(ee_ref[g], 0.0)
        e = jnp.maximum(dot(e, eW1_ref[...]) + eb1_ref[...], 0.0)
        e = jnp.maximum(dot(e, eW2_ref[...]) + eb2_ref[...], 0.0)
        es.append(e)
    gsum = gs[0] + gs[1]
    esum = es[0] + es[1]
    h = jnp.maximum(dot(gsum, dW1_ref[:GH, :]) + dot(esum, dW1_ref[GH:, :])
                    + db1_ref[...], 0.0)
    h = jnp.maximum(dot(h, dW2_ref[...]) + db2_ref[...], 0.0)
    o_ref[...] = dot(h, dW3_ref[...]) + db3_ref[...]


def _tc7(sums, cnt, ee, fcW, fcb, eW1, eb1, eW2, eb2,
         dW1, db1, dW2, db2, dW3, db3):
    return pl.pallas_call(
        _tc7_body,
        out_shape=jax.ShapeDtypeStruct((B, OUT), jnp.float32),
    )(sums, cnt, ee, fcW, fcb, eW1, eb1, eW2, eb2,
      dW1, db1, dW2, db2, dW3, db3)


def kernel(x1, edge_index1, ent1, batch1, x2, edge_index2, ent2, batch2,
           atom_emb, gW1, gb1, gW2, gb2, fcW, fcb,
           ent_emb, eW1, eb1, eW2, eb2,
           dW1, db1, dW2, db2, dW3, db3):
    f32 = jnp.float32
    i32 = jnp.int32

    xs = jnp.pad(jnp.stack([x1, x2]).astype(f32),
                 ((0, 0), (0, NP - N)))[..., None]
    batchf = jnp.pad(jnp.stack([batch1, batch2]).astype(f32),
                     ((0, 0), (0, NP - N)),
                     constant_values=float(B))[..., None]
    batchp = jnp.pad(jnp.stack([batch1, batch2]).astype(i32),
                     ((0, 0), (0, NP - N)),
                     constant_values=B).reshape(2, BROWS, 128)
    src = jnp.pad(jnp.stack([edge_index1[0], edge_index2[0]]).astype(i32),
                  ((0, 0), (0, EPAD)))
    dst = jnp.pad(jnp.stack([edge_index1[1], edge_index2[1]]).astype(i32),
                  ((0, 0), (0, EPAD)), constant_values=DEAD)
    dstp = dst.reshape(2, EROWSP, 128)
    offs = (jnp.arange(2, dtype=i32) * 2 * NP)[:, None, None] + \
           (jnp.arange(2, dtype=i32) * NP)[None, :, None]
    srcp = (src[:, None, :] + offs).reshape(2, 2, EROWSP, 128)
    entp = jnp.stack([ent1, ent2]).astype(i32).reshape(2, 4, 128)
    zeros1 = jnp.zeros((NP,), f32)
    zeros2 = jnp.zeros((128, 128), f32)
    aemb = jnp.pad(atom_emb, ((0, 5), (0, 0)))

    deg, cnt, ee = _sc_stats()(dstp, batchp, entp, ent_emb, zeros1)
    degc = deg[..., None]

    hs1 = _tc1(xs, degc, aemb, gW1)
    acc1 = _sc_edge()(srcp, dstp, hs1.reshape(4 * NP, 128), zeros2)
    hs2 = _tc3(acc1, hs1, degc, gb1.reshape(1, -1), gW2)
    acc2 = _sc_edge()(srcp, dstp, hs2.reshape(4 * NP, 128), zeros2)
    sums = _tc5(acc2, hs2, degc, gb2.reshape(1, -1), batchf)

    cnt512 = cnt[:, :B][..., None]
    return _tc7(sums, cnt512, ee, fcW, fcb.reshape(1, -1),
                eW1, eb1.reshape(1, -1), eW2, eb2.reshape(1, -1),
                dW1, db1.reshape(1, -1), dW2, db2.reshape(1, -1),
                dW3, db3.reshape(1, -1))


# speedup vs baseline: 1.8661x; 1.8661x over previous
import functools

import jax
import jax.numpy as jnp
from jax import lax
from jax.experimental import pallas as pl
from jax.experimental.pallas import tpu as pltpu
from jax.experimental.pallas import tpu_sc as plsc

N = 10000
NP = 10240
E = 320000
B = 512
VOCAB = 100000
ED = 128
GH = 256
OUT = 128

EROWS = E // 128
EROWSP = 2560
CH = EROWSP // 16
EPAD = EROWSP * 128 - E
DEAD = N + 16
NB = 512
NBLK = NP // NB
TPC = 16
DPT = NP // TPC
BROWS = NP // 128
CPAD = B + 128


def _sc_stats_body(dstp, batchp, entp, ent_emb, zeros1,
                   deg_out, cnt_out, ee_out,
                   degacc, cntacc, idxb, onesv, rows, entb):
    c = lax.axis_index("c")
    s = lax.axis_index("s")
    for i in range(8):
        onesv[pl.ds(i * 16, 16)] = jnp.ones((16,), jnp.float32)

    @pl.when(s == 0)
    def _():
        pltpu.sync_copy(zeros1, degacc)

    @pl.when(s == 1)
    def _():
        pltpu.sync_copy(zeros1.at[pl.ds(0, CPAD)], cntacc)

    plsc.subcore_barrier()

    def deg_step(k, _):
        r = s + TPC * k
        pltpu.sync_copy(dstp.at[c, r], idxb.at[0])
        pltpu.sync_copy(onesv, degacc.at[idxb.at[0]], add=True)
        return 0

    lax.fori_loop(0, EROWSP // TPC, deg_step, 0)

    def cnt_step(k, _):
        r = s + TPC * k
        pltpu.sync_copy(batchp.at[c, r], idxb.at[0])
        pltpu.sync_copy(onesv, cntacc.at[idxb.at[0]], add=True)
        return 0

    nitb = (BROWS - s + TPC - 1) // TPC
    lax.fori_loop(0, nitb, cnt_step, 0)

    @pl.when(s < 4)
    def _():
        pltpu.sync_copy(entp.at[c, s], entb)
        pltpu.sync_copy(ent_emb.at[entb], rows)
        pltpu.sync_copy(rows, ee_out.at[c, pl.ds(s * 128, 128)])

    plsc.subcore_barrier()
    pltpu.sync_copy(degacc.at[pl.ds(s * DPT, DPT)],
                    deg_out.at[c, pl.ds(s * DPT, DPT)])

    @pl.when(s == 0)
    def _():
        pltpu.sync_copy(cntacc, cnt_out.at[c])


@functools.lru_cache(maxsize=None)
def _sc_stats():
    return pl.kernel(
        _sc_stats_body,
        out_type=(
            jax.ShapeDtypeStruct((2, NP), jnp.float32),
            jax.ShapeDtypeStruct((2, CPAD), jnp.float32),
            jax.ShapeDtypeStruct((2, B, ED), jnp.float32),
        ),
        mesh=plsc.VectorSubcoreMesh(core_axis_name="c", subcore_axis_name="s"),
        scratch_types=[
            pltpu.VMEM_SHARED((NP,), jnp.float32),
            pltpu.VMEM_SHARED((CPAD,), jnp.float32),
            pltpu.VMEM((1, 128), jnp.int32),
            pltpu.VMEM((128,), jnp.float32),
            pltpu.VMEM((128, 128), jnp.float32),
            pltpu.VMEM((128,), jnp.int32),
        ],
    )


IBLK = 32


def _sc_edge_body(srcp, dstp, hs_all, zeros2,
                  acc_out,
                  accS, sidxb, didxb, rows0, rows1,
                  g0, g1, s0, s1):
    c = lax.axis_index("c")
    s = lax.axis_index("s")
    rows = (rows0, rows1)
    gsem = (g0, g1)
    ssem = (s0, s1)
    start = CH * s

    def gather_start(k, b):
        pltpu.async_copy(hs_all.at[sidxb.at[k]], rows[b], gsem[b])

    def gather_wait(k, b):
        pltpu.make_async_copy(hs_all.at[sidxb.at[k]], rows[b],
                              gsem[b]).wait()

    def scat_start(k, b):
        pltpu.async_copy(rows[b], accS.at[didxb.at[k]], ssem[b], add=True)

    def scat_wait(k, b):
        pltpu.make_async_copy(rows[b], accS.at[didxb.at[k]], ssem[b]).wait()

    def graph_pass(g, _):
        for z in range(DPT // 128):
            pltpu.sync_copy(zeros2,
                            accS.at[pl.ds(s * DPT + z * 128, 128)])
        plsc.subcore_barrier()

        def block(blk, _):
            base = start + blk * IBLK
            pltpu.sync_copy(srcp.at[g, c, pl.ds(base, IBLK)], sidxb)
            pltpu.sync_copy(dstp.at[g, pl.ds(base, IBLK)], didxb)

            def inner(k, _):
                pltpu.sync_copy(hs_all.at[sidxb.at[k]], rows0)
                pltpu.sync_copy(rows0, accS.at[didxb.at[k]], add=True)
                return 0

            lax.fori_loop(0, IBLK, inner, 0)
            return 0

        lax.fori_loop(0, CH // IBLK, block, 0)

        plsc.subcore_barrier()
        for z in range(DPT // 128):
            pltpu.sync_copy(accS.at[pl.ds(s * DPT + z * 128, 128)],
                            acc_out.at[g, c, pl.ds(s * DPT + z * 128, 128)])
        plsc.subcore_barrier()
        return 0

    lax.fori_loop(0, 2, graph_pass, 0)


@functools.lru_cache(maxsize=None)
def _sc_edge():
    return pl.kernel(
        _sc_edge_body,
        out_type=jax.ShapeDtypeStruct((2, 2, NP, 128), jnp.float32),
        mesh=plsc.VectorSubcoreMesh(core_axis_name="c", subcore_axis_name="s"),
        scratch_types=[
            pltpu.VMEM_SHARED((NP, 128), jnp.float32),
            pltpu.VMEM((IBLK, 128), jnp.int32),
            pltpu.VMEM((IBLK, 128), jnp.int32),
            pltpu.VMEM((128, 128), jnp.float32),
            pltpu.VMEM((128, 128), jnp.float32),
            pltpu.SemaphoreType.DMA,
            pltpu.SemaphoreType.DMA,
            pltpu.SemaphoreType.DMA,
            pltpu.SemaphoreType.DMA,
        ],
    )


def _tc1_body(xs_ref, deg_ref, aemb_ref, gW1_ref, hs_ref):
    x = xs_ref[0]
    iota = lax.broadcasted_iota(jnp.int32, (NB, 16), 1).astype(jnp.float32)
    onehot = (x == iota).astype(jnp.float32)
    weff = jnp.dot(aemb_ref[...], gW1_ref[...],
                   preferred_element_type=jnp.float32)
    h = jnp.dot(onehot, weff, preferred_element_type=jnp.float32)
    dinv = lax.rsqrt(deg_ref[0] + 1.0)
    hs = h * dinv
    hs_ref[0, 0] = hs[:, :128]
    hs_ref[0, 1] = hs[:, 128:]


def _tc1(xs, degc, aemb, gW1):
    return pl.pallas_call(
        _tc1_body,
        grid=(2, NBLK),
        in_specs=[
            pl.BlockSpec((1, NB, 1), lambda g, r: (g, r, 0)),
            pl.BlockSpec((1, NB, 1), lambda g, r: (g, r, 0)),
            pl.BlockSpec((16, ED), lambda g, r: (0, 0)),
            pl.BlockSpec((ED, GH), lambda g, r: (0, 0)),
        ],
        out_specs=pl.BlockSpec((1, 2, NB, 128), lambda g, r: (g, 0, r, 0)),
        out_shape=jax.ShapeDtypeStruct((2, 2, NP, 128), jnp.float32),
    )(xs, degc, aemb, gW1)


def _tc3_body(acc_ref, hs_ref, deg_ref, b1_ref, gW2_ref, out_ref):
    dinv = lax.rsqrt(deg_ref[0] + 1.0)
    a = jnp.concatenate([acc_ref[0, 0], acc_ref[0, 1]], axis=1)
    hsv = jnp.concatenate([hs_ref[0, 0], hs_ref[0, 1]], axis=1)
    out1 = jnp.maximum((a + hsv) * dinv + b1_ref[...], 0.0)
    h2 = jnp.dot(out1, gW2_ref[...], preferred_element_type=jnp.float32)
    h2s = h2 * dinv
    out_ref[0, 0] = h2s[:, :128]
    out_ref[0, 1] = h2s[:, 128:]


def _tc3(acc1, hs1, degc, b1, gW2):
    return pl.pallas_call(
        _tc3_body,
        grid=(2, NBLK),
        in_specs=[
            pl.BlockSpec((1, 2, NB, 128), lambda g, r: (g, 0, r, 0)),
            pl.BlockSpec((1, 2, NB, 128), lambda g, r: (g, 0, r, 0)),
            pl.BlockSpec((1, NB, 1), lambda g, r: (g, r, 0)),
            pl.BlockSpec((1, GH), lambda g, r: (0, 0)),
            pl.BlockSpec((GH, GH), lambda g, r: (0, 0)),
        ],
        out_specs=pl.BlockSpec((1, 2, NB, 128), lambda g, r: (g, 0, r, 0)),
        out_shape=jax.ShapeDtypeStruct((2, 2, NP, 128), jnp.float32),
    )(acc1, hs1, degc, b1, gW2)


def _tc5_body(acc_ref, hs_ref, deg_ref, b2_ref, batch_ref, sums_ref):
    r = pl.program_id(1)
    dinv = lax.rsqrt(deg_ref[0] + 1.0)
    a = jnp.concatenate([acc_ref[0, 0], acc_ref[0, 1]], axis=1)
    hsv = jnp.concatenate([hs_ref[0, 0], hs_ref[0, 1]], axis=1)
    out2 = jnp.maximum((a + hsv) * dinv + b2_ref[...], 0.0)
    iota = lax.broadcasted_iota(jnp.int32, (NB, B), 1).astype(jnp.float32)
    onehot = (batch_ref[0] == iota).astype(jnp.float32)
    part = lax.dot_general(onehot, out2, (((0,), (0,)), ((), ())),
                           preferred_element_type=jnp.float32)

    @pl.when(r == 0)
    def _():
        sums_ref[0] = jnp.zeros((B, GH), jnp.float32)

    sums_ref[0] += part


def _tc5(acc2, hs2, degc, b2, batchf):
    return pl.pallas_call(
        _tc5_body,
        grid=(2, NBLK),
        in_specs=[
            pl.BlockSpec((1, 2, NB, 128), lambda g, r: (g, 0, r, 0)),
            pl.BlockSpec((1, 2, NB, 128), lambda g, r: (g, 0, r, 0)),
            pl.BlockSpec((1, NB, 1), lambda g, r: (g, r, 0)),
            pl.BlockSpec((1, GH), lambda g, r: (0, 0)),
            pl.BlockSpec((1, NB, 1), lambda g, r: (g, r, 0)),
        ],
        out_specs=pl.BlockSpec((1, B, GH), lambda g, r: (g, 0, 0)),
        out_shape=jax.ShapeDtypeStruct((2, B, GH), jnp.float32),
    )(acc2, hs2, degc, b2, batchf)


def _tc7_body(sums_ref, cnt_ref, ee_ref, fcW_ref, fcb_ref,
              eW1_ref, eb1_ref, eW2_ref, eb2_ref,
              dW1_ref, db1_ref, dW2_ref, db2_ref, dW3_ref, db3_ref, o_ref):
    dot = functools.partial(jnp.dot, preferred_element_type=jnp.float32)
    gs = []
    es = []
    for g in range(2):
        pooled = sums_ref[g] / jnp.maximum(cnt_ref[g], 1.0)
        gs.append(dot(pooled, fcW_ref[...]) + fcb_ref[...])
        e = jnp.maximum(ee_ref[g], 0.0)
        e = jnp.maximum(dot(e, eW1_ref[...]) + eb1_ref[...], 0.0)
        e = jnp.maximum(dot(e, eW2_ref[...]) + eb2_ref[...], 0.0)
        es.append(e)
    gsum = gs[0] + gs[1]
    esum = es[0] + es[1]
    h = jnp.maximum(dot(gsum, dW1_ref[:GH, :]) + dot(esum, dW1_ref[GH:, :])
                    + db1_ref[...], 0.0)
    h = jnp.maximum(dot(h, dW2_ref[...]) + db2_ref[...], 0.0)
    o_ref[...] = dot(h, dW3_ref[...]) + db3_ref[...]


def _tc7(sums, cnt, ee, fcW, fcb, eW1, eb1, eW2, eb2,
         dW1, db1, dW2, db2, dW3, db3):
    return pl.pallas_call(
        _tc7_body,
        out_shape=jax.ShapeDtypeStruct((B, OUT), jnp.float32),
    )(sums, cnt, ee, fcW, fcb, eW1, eb1, eW2, eb2,
      dW1, db1, dW2, db2, dW3, db3)


def kernel(x1, edge_index1, ent1, batch1, x2, edge_index2, ent2, batch2,
           atom_emb, gW1, gb1, gW2, gb2, fcW, fcb,
           ent_emb, eW1, eb1, eW2, eb2,
           dW1, db1, dW2, db2, dW3, db3):
    f32 = jnp.float32
    i32 = jnp.int32

    xs = jnp.pad(jnp.stack([x1, x2]).astype(f32),
                 ((0, 0), (0, NP - N)))[..., None]
    batchf = jnp.pad(jnp.stack([batch1, batch2]).astype(f32),
                     ((0, 0), (0, NP - N)),
                     constant_values=float(B))[..., None]
    batchp = jnp.pad(jnp.stack([batch1, batch2]).astype(i32),
                     ((0, 0), (0, NP - N)),
                     constant_values=B).reshape(2, BROWS, 128)
    pad_src = jnp.broadcast_to((jnp.arange(EPAD, dtype=i32) * 79) % N,
                               (2, EPAD))
    pad_dst = jnp.broadcast_to(N + (jnp.arange(EPAD, dtype=i32) % (NP - N)),
                               (2, EPAD))
    src = jnp.concatenate(
        [jnp.stack([edge_index1[0], edge_index2[0]]).astype(i32), pad_src], 1)
    dst = jnp.concatenate(
        [jnp.stack([edge_index1[1], edge_index2[1]]).astype(i32), pad_dst], 1)
    dstp = dst.reshape(2, EROWSP, 128)
    offs = (jnp.arange(2, dtype=i32) * 2 * NP)[:, None, None] + \
           (jnp.arange(2, dtype=i32) * NP)[None, :, None]
    srcp = (src[:, None, :] + offs).reshape(2, 2, EROWSP, 128)
    entp = jnp.stack([ent1, ent2]).astype(i32).reshape(2, 4, 128)
    zeros1 = jnp.zeros((NP,), f32)
    zeros2 = jnp.zeros((128, 128), f32)
    aemb = jnp.pad(atom_emb, ((0, 5), (0, 0)))

    deg, cnt, ee = _sc_stats()(dstp, batchp, entp, ent_emb, zeros1)
    degc = deg[..., None]

    hs1 = _tc1(xs, degc, aemb, gW1)
    acc1 = _sc_edge()(srcp, dstp, hs1.reshape(4 * NP, 128), zeros2)
    hs2 = _tc3(acc1, hs1, degc, gb1.reshape(1, -1), gW2)
    acc2 = _sc_edge()(srcp, dstp, hs2.reshape(4 * NP, 128), zeros2)
    sums = _tc5(acc2, hs2, degc, gb2.reshape(1, -1), batchf)

    cnt512 = cnt[:, :B][..., None]
    return _tc7(sums, cnt512, ee, fcW, fcb.reshape(1, -1),
                eW1, eb1.reshape(1, -1), eW2, eb2.reshape(1, -1),
                dW1, db1.reshape(1, -1), dW2, db2.reshape(1, -1),
                dW3, db3.reshape(1, -1))


# speedup vs baseline: 2.3882x; 1.2798x over previous
import functools

import jax
import jax.numpy as jnp
from jax import lax
from jax.experimental import pallas as pl
from jax.experimental.pallas import tpu as pltpu
from jax.experimental.pallas import tpu_sc as plsc

N = 10000
NP = 10240
E = 320000
B = 512
VOCAB = 100000
ED = 128
GH = 256
OUT = 128

EROWS = E // 128
EROWSP = 2560
CH = EROWSP // 16
EPAD = EROWSP * 128 - E
DEAD = N + 16
NB = 512
NBLK = NP // NB
TPC = 16
DPT = NP // TPC
BROWS = NP // 128
CPAD = B + 128


def _sc_stats_body(dstp, batchp, entp, ent_emb, zeros1,
                   deg_out, cnt_out, ee_out,
                   degacc, cntacc, idxb, onesv, rows, entb):
    c = lax.axis_index("c")
    s = lax.axis_index("s")
    for i in range(8):
        onesv[pl.ds(i * 16, 16)] = jnp.ones((16,), jnp.float32)

    @pl.when(s == 0)
    def _():
        pltpu.sync_copy(zeros1, degacc)

    @pl.when(s == 1)
    def _():
        pltpu.sync_copy(zeros1.at[pl.ds(0, CPAD)], cntacc)

    plsc.subcore_barrier()

    def deg_step(k, _):
        r = s + TPC * k
        pltpu.sync_copy(dstp.at[c, r], idxb.at[0])
        pltpu.sync_copy(onesv, degacc.at[idxb.at[0]], add=True)
        return 0

    lax.fori_loop(0, EROWSP // TPC, deg_step, 0)

    def cnt_step(k, _):
        r = s + TPC * k
        pltpu.sync_copy(batchp.at[c, r], idxb.at[0])
        pltpu.sync_copy(onesv, cntacc.at[idxb.at[0]], add=True)
        return 0

    nitb = (BROWS - s + TPC - 1) // TPC
    lax.fori_loop(0, nitb, cnt_step, 0)

    @pl.when(s < 4)
    def _():
        pltpu.sync_copy(entp.at[c, s], entb)
        pltpu.sync_copy(ent_emb.at[entb], rows)
        pltpu.sync_copy(rows, ee_out.at[c, pl.ds(s * 128, 128)])

    plsc.subcore_barrier()
    pltpu.sync_copy(degacc.at[pl.ds(s * DPT, DPT)],
                    deg_out.at[c, pl.ds(s * DPT, DPT)])

    @pl.when(s == 0)
    def _():
        pltpu.sync_copy(cntacc, cnt_out.at[c])


@functools.lru_cache(maxsize=None)
def _sc_stats():
    return pl.kernel(
        _sc_stats_body,
        out_type=(
            jax.ShapeDtypeStruct((2, NP), jnp.float32),
            jax.ShapeDtypeStruct((2, CPAD), jnp.float32),
            jax.ShapeDtypeStruct((2, B, ED), jnp.float32),
        ),
        mesh=plsc.VectorSubcoreMesh(core_axis_name="c", subcore_axis_name="s"),
        scratch_types=[
            pltpu.VMEM_SHARED((NP,), jnp.float32),
            pltpu.VMEM_SHARED((CPAD,), jnp.float32),
            pltpu.VMEM((1, 128), jnp.int32),
            pltpu.VMEM((128,), jnp.float32),
            pltpu.VMEM((128, 128), jnp.float32),
            pltpu.VMEM((128,), jnp.int32),
        ],
    )


IBLK = 32


def _sc_edge_body(srcp, dstp, hs_all, zeros2,
                  acc_out,
                  accS, sidxb, didxb, rows0, rows1,
                  g0, g1, s0, s1):
    c = lax.axis_index("c")
    s = lax.axis_index("s")
    rows = (rows0, rows1)
    gsem = (g0, g1)
    ssem = (s0, s1)
    start = CH * s

    def gather_start(k, b):
        pltpu.async_copy(hs_all.at[sidxb.at[k]], rows[b], gsem[b])

    def gather_wait(k, b):
        pltpu.make_async_copy(hs_all.at[sidxb.at[k]], rows[b],
                              gsem[b]).wait()

    def scat_start(k, b):
        pltpu.async_copy(rows[b], accS.at[didxb.at[k]], ssem[b], add=True)

    def scat_wait(k, b):
        pltpu.make_async_copy(rows[b], accS.at[didxb.at[k]], ssem[b]).wait()

    def graph_pass(g, _):
        for z in range(DPT // 128):
            pltpu.sync_copy(zeros2,
                            accS.at[pl.ds(s * DPT + z * 128, 128)])
        plsc.subcore_barrier()

        def block(blk, _):
            base = start + blk * IBLK
            pltpu.sync_copy(srcp.at[g, c, pl.ds(base, IBLK)], sidxb)
            pltpu.sync_copy(dstp.at[g, pl.ds(base, IBLK)], didxb)
            gather_start(0, 0)

            def inner(i, _):
                for b in range(2):
                    k = 2 * i + b
                    nb = 1 - b

                    @pl.when(k >= 1)
                    def _():
                        scat_wait(k - 1, nb)
                    gather_wait(k, b)
                    scat_start(k, b)

                    @pl.when(k + 1 < IBLK)
                    def _():
                        gather_start(k + 1, nb)
                return 0

            lax.fori_loop(0, IBLK // 2, inner, 0)
            scat_wait(IBLK - 1, (IBLK - 1) % 2)
            return 0

        lax.fori_loop(0, CH // IBLK, block, 0)

        plsc.subcore_barrier()
        for z in range(DPT // 128):
            pltpu.sync_copy(accS.at[pl.ds(s * DPT + z * 128, 128)],
                            acc_out.at[g, c, pl.ds(s * DPT + z * 128, 128)])
        plsc.subcore_barrier()
        return 0

    lax.fori_loop(0, 2, graph_pass, 0)


@functools.lru_cache(maxsize=None)
def _sc_edge():
    return pl.kernel(
        _sc_edge_body,
        out_type=jax.ShapeDtypeStruct((2, 2, NP, 128), jnp.float32),
        mesh=plsc.VectorSubcoreMesh(core_axis_name="c", subcore_axis_name="s"),
        scratch_types=[
            pltpu.VMEM_SHARED((NP, 128), jnp.float32),
            pltpu.VMEM((IBLK, 128), jnp.int32),
            pltpu.VMEM((IBLK, 128), jnp.int32),
            pltpu.VMEM((128, 128), jnp.float32),
            pltpu.VMEM((128, 128), jnp.float32),
            pltpu.SemaphoreType.DMA,
            pltpu.SemaphoreType.DMA,
            pltpu.SemaphoreType.DMA,
            pltpu.SemaphoreType.DMA,
        ],
    )


def _tc1_body(xs_ref, deg_ref, aemb_ref, gW1_ref, hs_ref):
    x = xs_ref[0]
    iota = lax.broadcasted_iota(jnp.int32, (NB, 16), 1).astype(jnp.float32)
    onehot = (x == iota).astype(jnp.float32)
    weff = jnp.dot(aemb_ref[...], gW1_ref[...],
                   preferred_element_type=jnp.float32)
    h = jnp.dot(onehot, weff, preferred_element_type=jnp.float32)
    dinv = lax.rsqrt(deg_ref[0] + 1.0)
    hs = h * dinv
    hs_ref[0, 0] = hs[:, :128]
    hs_ref[0, 1] = hs[:, 128:]


def _tc1(xs, degc, aemb, gW1):
    return pl.pallas_call(
        _tc1_body,
        grid=(2, NBLK),
        in_specs=[
            pl.BlockSpec((1, NB, 1), lambda g, r: (g, r, 0)),
            pl.BlockSpec((1, NB, 1), lambda g, r: (g, r, 0)),
            pl.BlockSpec((16, ED), lambda g, r: (0, 0)),
            pl.BlockSpec((ED, GH), lambda g, r: (0, 0)),
        ],
        out_specs=pl.BlockSpec((1, 2, NB, 128), lambda g, r: (g, 0, r, 0)),
        out_shape=jax.ShapeDtypeStruct((2, 2, NP, 128), jnp.float32),
    )(xs, degc, aemb, gW1)


def _tc3_body(acc_ref, hs_ref, deg_ref, b1_ref, gW2_ref, out_ref):
    dinv = lax.rsqrt(deg_ref[0] + 1.0)
    a = jnp.concatenate([acc_ref[0, 0], acc_ref[0, 1]], axis=1)
    hsv = jnp.concatenate([hs_ref[0, 0], hs_ref[0, 1]], axis=1)
    out1 = jnp.maximum((a + hsv) * dinv + b1_ref[...], 0.0)
    h2 = jnp.dot(out1, gW2_ref[...], preferred_element_type=jnp.float32)
    h2s = h2 * dinv
    out_ref[0, 0] = h2s[:, :128]
    out_ref[0, 1] = h2s[:, 128:]


def _tc3(acc1, hs1, degc, b1, gW2):
    return pl.pallas_call(
        _tc3_body,
        grid=(2, NBLK),
        in_specs=[
            pl.BlockSpec((1, 2, NB, 128), lambda g, r: (g, 0, r, 0)),
            pl.BlockSpec((1, 2, NB, 128), lambda g, r: (g, 0, r, 0)),
            pl.BlockSpec((1, NB, 1), lambda g, r: (g, r, 0)),
            pl.BlockSpec((1, GH), lambda g, r: (0, 0)),
            pl.BlockSpec((GH, GH), lambda g, r: (0, 0)),
        ],
        out_specs=pl.BlockSpec((1, 2, NB, 128), lambda g, r: (g, 0, r, 0)),
        out_shape=jax.ShapeDtypeStruct((2, 2, NP, 128), jnp.float32),
    )(acc1, hs1, degc, b1, gW2)


def _tc5_body(acc_ref, hs_ref, deg_ref, b2_ref, batch_ref, sums_ref):
    r = pl.program_id(1)
    dinv = lax.rsqrt(deg_ref[0] + 1.0)
    a = jnp.concatenate([acc_ref[0, 0], acc_ref[0, 1]], axis=1)
    hsv = jnp.concatenate([hs_ref[0, 0], hs_ref[0, 1]], axis=1)
    out2 = jnp.maximum((a + hsv) * dinv + b2_ref[...], 0.0)
    iota = lax.broadcasted_iota(jnp.int32, (NB, B), 1).astype(jnp.float32)
    onehot = (batch_ref[0] == iota).astype(jnp.float32)
    part = lax.dot_general(onehot, out2, (((0,), (0,)), ((), ())),
                           preferred_element_type=jnp.float32)

    @pl.when(r == 0)
    def _():
        sums_ref[0] = jnp.zeros((B, GH), jnp.float32)

    sums_ref[0] += part


def _tc5(acc2, hs2, degc, b2, batchf):
    return pl.pallas_call(
        _tc5_body,
        grid=(2, NBLK),
        in_specs=[
            pl.BlockSpec((1, 2, NB, 128), lambda g, r: (g, 0, r, 0)),
            pl.BlockSpec((1, 2, NB, 128), lambda g, r: (g, 0, r, 0)),
            pl.BlockSpec((1, NB, 1), lambda g, r: (g, r, 0)),
            pl.BlockSpec((1, GH), lambda g, r: (0, 0)),
            pl.BlockSpec((1, NB, 1), lambda g, r: (g, r, 0)),
        ],
        out_specs=pl.BlockSpec((1, B, GH), lambda g, r: (g, 0, 0)),
        out_shape=jax.ShapeDtypeStruct((2, B, GH), jnp.float32),
    )(acc2, hs2, degc, b2, batchf)


def _tc7_body(sums_ref, cnt_ref, ee_ref, fcW_ref, fcb_ref,
              eW1_ref, eb1_ref, eW2_ref, eb2_ref,
              dW1_ref, db1_ref, dW2_ref, db2_ref, dW3_ref, db3_ref, o_ref):
    dot = functools.partial(jnp.dot, preferred_element_type=jnp.float32)
    gs = []
    es = []
    for g in range(2):
        pooled = sums_ref[g] / jnp.maximum(cnt_ref[g], 1.0)
        gs.append(dot(pooled, fcW_ref[...]) + fcb_ref[...])
        e = jnp.maximum(ee_ref[g], 0.0)
        e = jnp.maximum(dot(e, eW1_ref[...]) + eb1_ref[...], 0.0)
        e = jnp.maximum(dot(e, eW2_ref[...]) + eb2_ref[...], 0.0)
        es.append(e)
    gsum = gs[0] + gs[1]
    esum = es[0] + es[1]
    h = jnp.maximum(dot(gsum, dW1_ref[:GH, :]) + dot(esum, dW1_ref[GH:, :])
                    + db1_ref[...], 0.0)
    h = jnp.maximum(dot(h, dW2_ref[...]) + db2_ref[...], 0.0)
    o_ref[...] = dot(h, dW3_ref[...]) + db3_ref[...]


def _tc7(sums, cnt, ee, fcW, fcb, eW1, eb1, eW2, eb2,
         dW1, db1, dW2, db2, dW3, db3):
    return pl.pallas_call(
        _tc7_body,
        out_shape=jax.ShapeDtypeStruct((B, OUT), jnp.float32),
    )(sums, cnt, ee, fcW, fcb, eW1, eb1, eW2, eb2,
      dW1, db1, dW2, db2, dW3, db3)


def kernel(x1, edge_index1, ent1, batch1, x2, edge_index2, ent2, batch2,
           atom_emb, gW1, gb1, gW2, gb2, fcW, fcb,
           ent_emb, eW1, eb1, eW2, eb2,
           dW1, db1, dW2, db2, dW3, db3):
    f32 = jnp.float32
    i32 = jnp.int32

    xs = jnp.pad(jnp.stack([x1, x2]).astype(f32),
                 ((0, 0), (0, NP - N)))[..., None]
    batchf = jnp.pad(jnp.stack([batch1, batch2]).astype(f32),
                     ((0, 0), (0, NP - N)),
                     constant_values=float(B))[..., None]
    batchp = jnp.pad(jnp.stack([batch1, batch2]).astype(i32),
                     ((0, 0), (0, NP - N)),
                     constant_values=B).reshape(2, BROWS, 128)
    pad_src = jnp.broadcast_to((jnp.arange(EPAD, dtype=i32) * 79) % N,
                               (2, EPAD))
    pad_dst = jnp.broadcast_to(N + (jnp.arange(EPAD, dtype=i32) % (NP - N)),
                               (2, EPAD))
    src = jnp.concatenate(
        [jnp.stack([edge_index1[0], edge_index2[0]]).astype(i32), pad_src], 1)
    dst = jnp.concatenate(
        [jnp.stack([edge_index1[1], edge_index2[1]]).astype(i32), pad_dst], 1)
    dstp = dst.reshape(2, EROWSP, 128)
    offs = (jnp.arange(2, dtype=i32) * 2 * NP)[:, None, None] + \
           (jnp.arange(2, dtype=i32) * NP)[None, :, None]
    srcp = (src[:, None, :] + offs).reshape(2, 2, EROWSP, 128)
    entp = jnp.stack([ent1, ent2]).astype(i32).reshape(2, 4, 128)
    zeros1 = jnp.zeros((NP,), f32)
    zeros2 = jnp.zeros((128, 128), f32)
    aemb = jnp.pad(atom_emb, ((0, 5), (0, 0)))

    deg, cnt, ee = _sc_stats()(dstp, batchp, entp, ent_emb, zeros1)
    degc = deg[..., None]

    hs1 = _tc1(xs, degc, aemb, gW1)
    acc1 = _sc_edge()(srcp, dstp, hs1.reshape(4 * NP, 128), zeros2)
    hs2 = _tc3(acc1, hs1, degc, gb1.reshape(1, -1), gW2)
    acc2 = _sc_edge()(srcp, dstp, hs2.reshape(4 * NP, 128), zeros2)
    sums = _tc5(acc2, hs2, degc, gb2.reshape(1, -1), batchf)

    cnt512 = cnt[:, :B][..., None]
    return _tc7(sums, cnt512, ee, fcW, fcb.reshape(1, -1),
                eW1, eb1.reshape(1, -1), eW2, eb2.reshape(1, -1),
                dW1, db1.reshape(1, -1), dW2, db2.reshape(1, -1),
                dW3, db3.reshape(1, -1))


# speedup vs baseline: 2.7765x; 1.1626x over previous
import functools

import jax
import jax.numpy as jnp
from jax import lax
from jax.experimental import pallas as pl
from jax.experimental.pallas import tpu as pltpu
from jax.experimental.pallas import tpu_sc as plsc

N = 10000
NP = 10240
E = 320000
B = 512
VOCAB = 100000
ED = 128
GH = 256
OUT = 128

EROWS = E // 128
EROWSP = 2560
CH = EROWSP // 16
EPAD = EROWSP * 128 - E
DEAD = N + 16
NB = 512
NBLK = NP // NB
TPC = 16
DPT = NP // TPC
BROWS = NP // 128
CPAD = B + 128


def _sc_stats_body(dstp, batchp, entp, ent_emb, zeros1,
                   deg_out, cnt_out, ee_out,
                   degacc, cntacc, idxb, idxb2, onesv, rows, entb,
                   ia0, ia1):
    c = lax.axis_index("c")
    s = lax.axis_index("s")
    ib = (idxb, idxb2)
    isem = (ia0, ia1)
    for i in range(8):
        onesv[pl.ds(i * 16, 16)] = jnp.ones((16,), jnp.float32)

    @pl.when(s == 0)
    def _():
        pltpu.sync_copy(zeros1, degacc)

    @pl.when(s == 1)
    def _():
        pltpu.sync_copy(zeros1.at[pl.ds(0, CPAD)], cntacc)

    plsc.subcore_barrier()

    NIT = EROWSP // TPC

    def idx_load(k, b):
        pltpu.async_copy(dstp.at[c, s + TPC * k], ib[b].at[0], isem[b])

    def idx_wt(k, b):
        pltpu.make_async_copy(dstp.at[c, s + TPC * k], ib[b].at[0],
                              isem[b]).wait()

    pltpu.sync_copy(dstp.at[c, s], idxb.at[0])

    def deg_step(i, _):
        for b in range(2):
            k = 2 * i + b
            nb = 1 - b

            @pl.when(k + 1 < NIT)
            def _():
                idx_load(k + 1, nb)
            pltpu.sync_copy(onesv, degacc.at[ib[b].at[0]], add=True)

            @pl.when(k + 1 < NIT)
            def _():
                idx_wt(k + 1, nb)
        return 0

    lax.fori_loop(0, NIT // 2, deg_step, 0)

    def cnt_step(k, _):
        r = s + TPC * k
        pltpu.sync_copy(batchp.at[c, r], idxb.at[0])
        pltpu.sync_copy(onesv, cntacc.at[idxb.at[0]], add=True)
        return 0

    nitb = (BROWS - s + TPC - 1) // TPC
    lax.fori_loop(0, nitb, cnt_step, 0)

    @pl.when(s < 4)
    def _():
        pltpu.sync_copy(entp.at[c, s], entb)
        pltpu.sync_copy(ent_emb.at[entb], rows)
        pltpu.sync_copy(rows, ee_out.at[c, pl.ds(s * 128, 128)])

    plsc.subcore_barrier()
    pltpu.sync_copy(degacc.at[pl.ds(s * DPT, DPT)],
                    deg_out.at[c, pl.ds(s * DPT, DPT)])

    @pl.when(s == 0)
    def _():
        pltpu.sync_copy(cntacc, cnt_out.at[c])


@functools.lru_cache(maxsize=None)
def _sc_stats():
    return pl.kernel(
        _sc_stats_body,
        out_type=(
            jax.ShapeDtypeStruct((2, NP), jnp.float32),
            jax.ShapeDtypeStruct((2, CPAD), jnp.float32),
            jax.ShapeDtypeStruct((2, B, ED), jnp.float32),
        ),
        mesh=plsc.VectorSubcoreMesh(core_axis_name="c", subcore_axis_name="s"),
        scratch_types=[
            pltpu.VMEM_SHARED((NP,), jnp.float32),
            pltpu.VMEM_SHARED((CPAD,), jnp.float32),
            pltpu.VMEM((1, 128), jnp.int32),
            pltpu.VMEM((1, 128), jnp.int32),
            pltpu.VMEM((128,), jnp.float32),
            pltpu.VMEM((128, 128), jnp.float32),
            pltpu.VMEM((128,), jnp.int32),
            pltpu.SemaphoreType.DMA,
            pltpu.SemaphoreType.DMA,
        ],
    )


IBLK = 32


def _sc_edge_body(srcp, dstp, hs_all, zeros2,
                  acc_out,
                  accS, sidxb, didxb, rows0, rows1,
                  g0, g1, s0, s1):
    c = lax.axis_index("c")
    s = lax.axis_index("s")
    rows = (rows0, rows1)
    gsem = (g0, g1)
    ssem = (s0, s1)
    start = CH * s

    def gather_start(k, b):
        pltpu.async_copy(hs_all.at[sidxb.at[k]], rows[b], gsem[b])

    def gather_wait(k, b):
        pltpu.make_async_copy(hs_all.at[sidxb.at[k]], rows[b],
                              gsem[b]).wait()

    def scat_start(k, b):
        pltpu.async_copy(rows[b], accS.at[didxb.at[k]], ssem[b], add=True)

    def scat_wait(k, b):
        pltpu.make_async_copy(rows[b], accS.at[didxb.at[k]], ssem[b]).wait()

    def graph_pass(g, _):
        for z in range(DPT // 128):
            pltpu.sync_copy(zeros2,
                            accS.at[pl.ds(s * DPT + z * 128, 128)])
        plsc.subcore_barrier()

        def block(blk, _):
            base = start + blk * IBLK
            pltpu.sync_copy(srcp.at[g, c, pl.ds(base, IBLK)], sidxb)
            pltpu.sync_copy(dstp.at[g, pl.ds(base, IBLK)], didxb)
            gather_start(0, 0)

            def inner(i, _):
                for b in range(2):
                    k = 2 * i + b
                    nb = 1 - b

                    @pl.when(k >= 1)
                    def _():
                        scat_wait(k - 1, nb)

                    @pl.when(k + 1 < IBLK)
                    def _():
                        gather_start(k + 1, nb)
                    gather_wait(k, b)
                    scat_start(k, b)
                return 0

            lax.fori_loop(0, IBLK // 2, inner, 0)
            scat_wait(IBLK - 1, (IBLK - 1) % 2)
            return 0

        lax.fori_loop(0, CH // IBLK, block, 0)

        plsc.subcore_barrier()
        for z in range(DPT // 128):
            pltpu.sync_copy(accS.at[pl.ds(s * DPT + z * 128, 128)],
                            acc_out.at[g, c, pl.ds(s * DPT + z * 128, 128)])
        plsc.subcore_barrier()
        return 0

    lax.fori_loop(0, 2, graph_pass, 0)


@functools.lru_cache(maxsize=None)
def _sc_edge():
    return pl.kernel(
        _sc_edge_body,
        out_type=jax.ShapeDtypeStruct((2, 2, NP, 128), jnp.float32),
        mesh=plsc.VectorSubcoreMesh(core_axis_name="c", subcore_axis_name="s"),
        scratch_types=[
            pltpu.VMEM_SHARED((NP, 128), jnp.float32),
            pltpu.VMEM((IBLK, 128), jnp.int32),
            pltpu.VMEM((IBLK, 128), jnp.int32),
            pltpu.VMEM((128, 128), jnp.float32),
            pltpu.VMEM((128, 128), jnp.float32),
            pltpu.SemaphoreType.DMA,
            pltpu.SemaphoreType.DMA,
            pltpu.SemaphoreType.DMA,
            pltpu.SemaphoreType.DMA,
        ],
    )


def _tc1_body(xs_ref, deg_ref, aemb_ref, gW1_ref, hs_ref):
    x = xs_ref[0]
    iota = lax.broadcasted_iota(jnp.int32, (NB, 16), 1).astype(jnp.float32)
    onehot = (x == iota).astype(jnp.float32)
    weff = jnp.dot(aemb_ref[...], gW1_ref[...],
                   preferred_element_type=jnp.float32)
    h = jnp.dot(onehot, weff, preferred_element_type=jnp.float32)
    dinv = lax.rsqrt(deg_ref[0] + 1.0)
    hs = h * dinv
    hs_ref[0, 0] = hs[:, :128]
    hs_ref[0, 1] = hs[:, 128:]


def _tc1(xs, degc, aemb, gW1):
    return pl.pallas_call(
        _tc1_body,
        grid=(2, NBLK),
        in_specs=[
            pl.BlockSpec((1, NB, 1), lambda g, r: (g, r, 0)),
            pl.BlockSpec((1, NB, 1), lambda g, r: (g, r, 0)),
            pl.BlockSpec((16, ED), lambda g, r: (0, 0)),
            pl.BlockSpec((ED, GH), lambda g, r: (0, 0)),
        ],
        out_specs=pl.BlockSpec((1, 2, NB, 128), lambda g, r: (g, 0, r, 0)),
        out_shape=jax.ShapeDtypeStruct((2, 2, NP, 128), jnp.float32),
    )(xs, degc, aemb, gW1)


def _tc3_body(acc_ref, hs_ref, deg_ref, b1_ref, gW2_ref, out_ref):
    dinv = lax.rsqrt(deg_ref[0] + 1.0)
    a = jnp.concatenate([acc_ref[0, 0], acc_ref[0, 1]], axis=1)
    hsv = jnp.concatenate([hs_ref[0, 0], hs_ref[0, 1]], axis=1)
    out1 = jnp.maximum((a + hsv) * dinv + b1_ref[...], 0.0)
    h2 = jnp.dot(out1, gW2_ref[...], preferred_element_type=jnp.float32)
    h2s = h2 * dinv
    out_ref[0, 0] = h2s[:, :128]
    out_ref[0, 1] = h2s[:, 128:]


def _tc3(acc1, hs1, degc, b1, gW2):
    return pl.pallas_call(
        _tc3_body,
        grid=(2, NBLK),
        in_specs=[
            pl.BlockSpec((1, 2, NB, 128), lambda g, r: (g, 0, r, 0)),
            pl.BlockSpec((1, 2, NB, 128), lambda g, r: (g, 0, r, 0)),
            pl.BlockSpec((1, NB, 1), lambda g, r: (g, r, 0)),
            pl.BlockSpec((1, GH), lambda g, r: (0, 0)),
            pl.BlockSpec((GH, GH), lambda g, r: (0, 0)),
        ],
        out_specs=pl.BlockSpec((1, 2, NB, 128), lambda g, r: (g, 0, r, 0)),
        out_shape=jax.ShapeDtypeStruct((2, 2, NP, 128), jnp.float32),
    )(acc1, hs1, degc, b1, gW2)


def _tc5_body(acc_ref, hs_ref, deg_ref, b2_ref, batch_ref, sums_ref):
    r = pl.program_id(1)
    dinv = lax.rsqrt(deg_ref[0] + 1.0)
    a = jnp.concatenate([acc_ref[0, 0], acc_ref[0, 1]], axis=1)
    hsv = jnp.concatenate([hs_ref[0, 0], hs_ref[0, 1]], axis=1)
    out2 = jnp.maximum((a + hsv) * dinv + b2_ref[...], 0.0)
    iota = lax.broadcasted_iota(jnp.int32, (NB, B), 1).astype(jnp.float32)
    onehot = (batch_ref[0] == iota).astype(jnp.float32)
    part = lax.dot_general(onehot, out2, (((0,), (0,)), ((), ())),
                           preferred_element_type=jnp.float32)

    @pl.when(r == 0)
    def _():
        sums_ref[0] = jnp.zeros((B, GH), jnp.float32)

    sums_ref[0] += part


def _tc5(acc2, hs2, degc, b2, batchf):
    return pl.pallas_call(
        _tc5_body,
        grid=(2, NBLK),
        in_specs=[
            pl.BlockSpec((1, 2, NB, 128), lambda g, r: (g, 0, r, 0)),
            pl.BlockSpec((1, 2, NB, 128), lambda g, r: (g, 0, r, 0)),
            pl.BlockSpec((1, NB, 1), lambda g, r: (g, r, 0)),
            pl.BlockSpec((1, GH), lambda g, r: (0, 0)),
            pl.BlockSpec((1, NB, 1), lambda g, r: (g, r, 0)),
        ],
        out_specs=pl.BlockSpec((1, B, GH), lambda g, r: (g, 0, 0)),
        out_shape=jax.ShapeDtypeStruct((2, B, GH), jnp.float32),
    )(acc2, hs2, degc, b2, batchf)


def _tc7_body(sums_ref, cnt_ref, ee_ref, fcW_ref, fcb_ref,
              eW1_ref, eb1_ref, eW2_ref, eb2_ref,
              dW1_ref, db1_ref, dW2_ref, db2_ref, dW3_ref, db3_ref, o_ref):
    dot = functools.partial(jnp.dot, preferred_element_type=jnp.float32)
    gs = []
    es = []
    for g in range(2):
        pooled = sums_ref[g] / jnp.maximum(cnt_ref[g], 1.0)
        gs.append(dot(pooled, fcW_ref[...]) + fcb_ref[...])
        e = jnp.maximum(ee_ref[g], 0.0)
        e = jnp.maximum(dot(e, eW1_ref[...]) + eb1_ref[...], 0.0)
        e = jnp.maximum(dot(e, eW2_ref[...]) + eb2_ref[...], 0.0)
        es.append(e)
    gsum = gs[0] + gs[1]
    esum = es[0] + es[1]
    h = jnp.maximum(dot(gsum, dW1_ref[:GH, :]) + dot(esum, dW1_ref[GH:, :])
                    + db1_ref[...], 0.0)
    h = jnp.maximum(dot(h, dW2_ref[...]) + db2_ref[...], 0.0)
    o_ref[...] = dot(h, dW3_ref[...]) + db3_ref[...]


def _tc7(sums, cnt, ee, fcW, fcb, eW1, eb1, eW2, eb2,
         dW1, db1, dW2, db2, dW3, db3):
    return pl.pallas_call(
        _tc7_body,
        out_shape=jax.ShapeDtypeStruct((B, OUT), jnp.float32),
    )(sums, cnt, ee, fcW, fcb, eW1, eb1, eW2, eb2,
      dW1, db1, dW2, db2, dW3, db3)


def kernel(x1, edge_index1, ent1, batch1, x2, edge_index2, ent2, batch2,
           atom_emb, gW1, gb1, gW2, gb2, fcW, fcb,
           ent_emb, eW1, eb1, eW2, eb2,
           dW1, db1, dW2, db2, dW3, db3):
    f32 = jnp.float32
    i32 = jnp.int32

    xs = jnp.pad(jnp.stack([x1, x2]).astype(f32),
                 ((0, 0), (0, NP - N)))[..., None]
    batchf = jnp.pad(jnp.stack([batch1, batch2]).astype(f32),
                     ((0, 0), (0, NP - N)),
                     constant_values=float(B))[..., None]
    batchp = jnp.pad(jnp.stack([batch1, batch2]).astype(i32),
                     ((0, 0), (0, NP - N)),
                     constant_values=B).reshape(2, BROWS, 128)
    pad_src = jnp.broadcast_to((jnp.arange(EPAD, dtype=i32) * 79) % N,
                               (2, EPAD))
    pad_dst = jnp.broadcast_to(N + (jnp.arange(EPAD, dtype=i32) % (NP - N)),
                               (2, EPAD))
    src = jnp.concatenate(
        [jnp.stack([edge_index1[0], edge_index2[0]]).astype(i32), pad_src], 1)
    dst = jnp.concatenate(
        [jnp.stack([edge_index1[1], edge_index2[1]]).astype(i32), pad_dst], 1)
    dstp = dst.reshape(2, EROWSP, 128)
    offs = (jnp.arange(2, dtype=i32) * 2 * NP)[:, None, None] + \
           (jnp.arange(2, dtype=i32) * NP)[None, :, None]
    srcp = (src[:, None, :] + offs).reshape(2, 2, EROWSP, 128)
    entp = jnp.stack([ent1, ent2]).astype(i32).reshape(2, 4, 128)
    zeros1 = jnp.zeros((NP,), f32)
    zeros2 = jnp.zeros((128, 128), f32)
    aemb = jnp.pad(atom_emb, ((0, 5), (0, 0)))

    deg, cnt, ee = _sc_stats()(dstp, batchp, entp, ent_emb, zeros1)
    degc = deg[..., None]

    hs1 = _tc1(xs, degc, aemb, gW1)
    acc1 = _sc_edge()(srcp, dstp, hs1.reshape(4 * NP, 128), zeros2)
    hs2 = _tc3(acc1, hs1, degc, gb1.reshape(1, -1), gW2)
    acc2 = _sc_edge()(srcp, dstp, hs2.reshape(4 * NP, 128), zeros2)
    sums = _tc5(acc2, hs2, degc, gb2.reshape(1, -1), batchf)

    cnt512 = cnt[:, :B][..., None]
    return _tc7(sums, cnt512, ee, fcW, fcb.reshape(1, -1),
                eW1, eb1.reshape(1, -1), eW2, eb2.reshape(1, -1),
                dW1, db1.reshape(1, -1), dW2, db2.reshape(1, -1),
                dW3, db3.reshape(1, -1))
